# baseline (speedup 1.0000x reference)
import jax
import jax.numpy as jnp
from jax.experimental import pallas as pl

HID = 768
FFN_D = 3072
NE = 8
TBLK = 512
NT = 2048 // TBLK


def _moe_dense_body(x_ref, wg_ref, w1_ref, b1_ref, w2_ref, b2_ref, out_ref):
    e = pl.program_id(0)
    ti = pl.program_id(1)
    x = x_ref[pl.ds(ti * TBLK, TBLK), :]
    logits = jnp.dot(x, wg_ref[...], preferred_element_type=jnp.float32)
    probs = jax.nn.softmax(logits, axis=-1)
    iota = jax.lax.broadcasted_iota(jnp.int32, probs.shape, 1)
    m1 = jnp.max(probs, axis=1, keepdims=True)
    i1 = jnp.min(jnp.where(probs == m1, iota, NE), axis=1, keepdims=True)
    pm = jnp.where(iota == i1, -1.0, probs)
    m2 = jnp.max(pm, axis=1, keepdims=True)
    i2 = jnp.min(jnp.where(pm == m2, iota, NE), axis=1, keepdims=True)
    denom = m1 + m2 + 1e-9
    ge = jnp.where(i1 == e, m1 / denom, 0.0) + jnp.where(i2 == e, m2 / denom, 0.0)
    ge = jnp.where(e == NE, 0.1, ge)

    h = jnp.dot(x, w1_ref[0], preferred_element_type=jnp.float32) + b1_ref[0]
    h = h * jax.nn.sigmoid(h)
    y = jnp.dot(h, w2_ref[0], preferred_element_type=jnp.float32) + b2_ref[0]
    contrib = ge * y

    @pl.when(e == 0)
    def _():
        out_ref[pl.ds(ti * TBLK, TBLK), :] = contrib

    @pl.when(e > 0)
    def _():
        out_ref[pl.ds(ti * TBLK, TBLK), :] += contrib


def _moe_dense(x, Wg, W1c, b1c, W2c, b2c):
    return pl.pallas_call(
        _moe_dense_body,
        grid=(NE + 1, NT),
        in_specs=[
            pl.BlockSpec((2048, HID), lambda e, t: (0, 0)),
            pl.BlockSpec((HID, NE), lambda e, t: (0, 0)),
            pl.BlockSpec((1, HID, FFN_D), lambda e, t: (e, 0, 0)),
            pl.BlockSpec((1, 1, FFN_D), lambda e, t: (e, 0, 0)),
            pl.BlockSpec((1, FFN_D, HID), lambda e, t: (e, 0, 0)),
            pl.BlockSpec((1, 1, HID), lambda e, t: (e, 0, 0)),
        ],
        out_specs=pl.BlockSpec((2048, HID), lambda e, t: (0, 0)),
        out_shape=jax.ShapeDtypeStruct((2048, HID), jnp.float32),
    )(x, Wg, W1c, b1c, W2c, b2c)


def kernel(hidden_states, Wg, W1, b1, W2, b2, Ws1, bs1, Ws2, bs2):
    orig = hidden_states.shape
    x = hidden_states.reshape(-1, orig[-1])
    W1c = jnp.concatenate([W1, Ws1[None]], axis=0)
    W2c = jnp.concatenate([W2, Ws2[None]], axis=0)
    b1c = jnp.concatenate([b1, bs1[None]], axis=0)[:, None, :]
    b2c = jnp.concatenate([b2, bs2[None]], axis=0)[:, None, :]
    out = _moe_dense(x, Wg, W1c, b1c, W2c, b2c)
    return out.reshape(orig)
